# baseline (speedup 1.0000x reference)
import functools

import jax
import jax.numpy as jnp
from jax import lax
from jax.experimental import pallas as pl
from jax.experimental.pallas import tpu as pltpu
from jax.experimental.pallas import tpu_sc as plsc

N = 10000
E = 320000
HID = 64
EPS = 1e-6

NC = 2
NS = 16
NW = NC * NS
LANES = 16

CH1 = 512
SUB = 128
NCHUNKS1 = E // CH1
TRIP1 = (NCHUNKS1 + NW - 1) // NW


def _s1_body(b1h, b2h, b3e, srcv, dstv, t_out, stats_out,
             idx_s, idx_d, g1, g2, b3, stats_v, sem):
    c = lax.axis_index("c")
    s = lax.axis_index("s")
    wid = s * NC + c

    for k in range(4):
        stats_v[0, pl.ds(16 * k, 16)] = jnp.zeros((16,), jnp.float32)
        stats_v[1, pl.ds(16 * k, 16)] = jnp.zeros((16,), jnp.float32)

    def chunk(i, carry):
        ci = wid + NW * i

        @pl.when(ci < NCHUNKS1)
        def _():
            base = ci * CH1
            for j in range(CH1 // SUB):
                pltpu.sync_copy(srcv.at[pl.ds(base + SUB * j, SUB)], idx_s.at[j])
                pltpu.sync_copy(dstv.at[pl.ds(base + SUB * j, SUB)], idx_d.at[j])
            copies = []
            for j in range(CH1 // SUB):
                copies.append(pltpu.async_copy(
                    b1h.at[idx_s.at[j]], g1.at[pl.ds(SUB * j, SUB)], sem))
                copies.append(pltpu.async_copy(
                    b2h.at[idx_d.at[j]], g2.at[pl.ds(SUB * j, SUB)], sem))
            pltpu.sync_copy(b3e.at[pl.ds(base, CH1), :], b3)
            for cp in copies:
                cp.wait()

            def edge(e, carry):
                acc = list(carry)
                for k in range(4):
                    v = g1[e, pl.ds(16 * k, 16)] + g2[e, pl.ds(16 * k, 16)] \
                        + b3[e, pl.ds(16 * k, 16)]
                    b3[e, pl.ds(16 * k, 16)] = v
                    acc[k] = acc[k] + v
                    acc[4 + k] = acc[4 + k] + v * v
                return tuple(acc)

            init = tuple(jnp.zeros((16,), jnp.float32) for _ in range(8))
            acc = lax.fori_loop(0, CH1, edge, init)
            for k in range(4):
                stats_v[0, pl.ds(16 * k, 16)] += acc[k]
                stats_v[1, pl.ds(16 * k, 16)] += acc[4 + k]
            pltpu.sync_copy(b3, t_out.at[pl.ds(base, CH1), :])
        return carry

    lax.fori_loop(0, TRIP1, chunk, 0)
    pltpu.sync_copy(stats_v, stats_out.at[wid])


CH2 = 256
NCHUNKS2 = E // CH2
TRIP2S = (NCHUNKS2 + NW - 1) // NW
NSTRIPE = N // NS


NHALF = N // NC
NSENT = 8
NACC = 5136
ZROWS = NACC // NS


def _s2_body(t_in, e_in, tbl, gidx, sidx, bnp,
             e_new, acc_h_out, acc_s_out,
             idx_g, idx_sc, g, tb, eb, sg, bn_v, zb, acc_h, acc_s, sem):
    c = lax.axis_index("c")
    s = lax.axis_index("s")
    wid = s * NC + c
    nbase = c * NHALF

    def zrow(r, carry):
        for k in range(4):
            zb[r, pl.ds(16 * k, 16)] = jnp.zeros((16,), jnp.float32)
        return carry
    lax.fori_loop(0, ZROWS, zrow, 0)
    pltpu.sync_copy(zb, acc_h.at[pl.ds(ZROWS * s, ZROWS), :])
    pltpu.sync_copy(zb, acc_s.at[pl.ds(ZROWS * s, ZROWS), :])
    plsc.subcore_barrier()

    pltpu.sync_copy(bnp, bn_v)
    av = [bn_v[0, pl.ds(16 * k, 16)] for k in range(4)]
    bv = [bn_v[1, pl.ds(16 * k, 16)] for k in range(4)]
    lane = lax.iota(jnp.int32, 16)
    sent = NHALF + NSENT * s + (lane & (NSENT - 1))

    def chunk(i, carry):
        ci = wid + NW * i

        @pl.when(ci < NCHUNKS2)
        def _():
            base = ci * CH2
            for j in range(CH2 // SUB):
                pltpu.sync_copy(gidx.at[pl.ds(base + SUB * j, SUB)],
                                idx_g.at[j])
                pltpu.sync_copy(sidx.at[pl.ds(base + SUB * j, SUB)],
                                idx_sc.at[j])
            copies = [pltpu.async_copy(tbl.at[idx_g.at[j]],
                                       g.at[pl.ds(SUB * j, SUB)], sem)
                      for j in range(CH2 // SUB)]
            pltpu.sync_copy(t_in.at[pl.ds(base, CH2), :], tb)
            pltpu.sync_copy(e_in.at[pl.ds(base, CH2), :], eb)

            def remap(r, cr):
                for j in range(CH2 // SUB):
                    iv = idx_sc[j, pl.ds(16 * r, 16)] - nbase
                    inb = (iv >= 0) & (iv < NHALF)
                    idx_sc[j, pl.ds(16 * r, 16)] = jnp.where(inb, iv, sent)
                return cr
            lax.fori_loop(0, SUB // 16, remap, 0)
            for cp in copies:
                cp.wait()

            def edge(ei, cr):
                for k in range(4):
                    u = jnp.maximum(
                        tb[ei, pl.ds(16 * k, 16)] * av[k] + bv[k], 0.0)
                    en = eb[ei, pl.ds(16 * k, 16)] + u
                    eb[ei, pl.ds(16 * k, 16)] = en
                    sig = 1.0 / (1.0 + jnp.exp(-en))
                    sg[ei, pl.ds(16 * k, 16)] = sig
                    g[ei, pl.ds(16 * k, 16)] = sig * g[ei, pl.ds(16 * k, 16)]
                return cr
            lax.fori_loop(0, CH2, edge, 0)

            pltpu.sync_copy(eb, e_new.at[pl.ds(base, CH2), :])
            for j in range(CH2 // SUB):
                pltpu.sync_copy(g.at[pl.ds(SUB * j, SUB)],
                                acc_h.at[idx_sc.at[j]], add=True)
                pltpu.sync_copy(sg.at[pl.ds(SUB * j, SUB)],
                                acc_s.at[idx_sc.at[j]], add=True)
        return carry
    lax.fori_loop(0, TRIP2S, chunk, 0)

    plsc.subcore_barrier()

    @pl.when(s < NS // NC)
    def _():
        rows = pl.ds(NSTRIPE * s, NSTRIPE)
        orows = pl.ds(nbase + NSTRIPE * s, NSTRIPE)
        pltpu.sync_copy(acc_h.at[rows, :], acc_h_out.at[orows, :])
        pltpu.sync_copy(acc_s.at[rows, :], acc_s_out.at[orows, :])


_s2_call = functools.partial(
    pl.kernel,
    out_type=(jax.ShapeDtypeStruct((E, HID), jnp.float32),
              jax.ShapeDtypeStruct((N, HID), jnp.float32),
              jax.ShapeDtypeStruct((N, HID), jnp.float32)),
    mesh=plsc.VectorSubcoreMesh(core_axis_name="c", subcore_axis_name="s"),
    compiler_params=pltpu.CompilerParams(use_tc_tiling_on_sc=False),
    scratch_types=[
        pltpu.VMEM((CH2 // SUB, SUB), jnp.int32),
        pltpu.VMEM((CH2 // SUB, SUB), jnp.int32),
        pltpu.VMEM((CH2, HID), jnp.float32),
        pltpu.VMEM((CH2, HID), jnp.float32),
        pltpu.VMEM((CH2, HID), jnp.float32),
        pltpu.VMEM((CH2, HID), jnp.float32),
        pltpu.VMEM((2, HID), jnp.float32),
        pltpu.VMEM((ZROWS, HID), jnp.float32),
        pltpu.VMEM_SHARED((NACC, HID), jnp.float32),
        pltpu.VMEM_SHARED((NACC, HID), jnp.float32),
        pltpu.SemaphoreType.DMA,
    ],
)(_s2_body)


_s1_call = functools.partial(
    pl.kernel,
    out_type=(jax.ShapeDtypeStruct((E, HID), jnp.float32),
              jax.ShapeDtypeStruct((NW, 2, HID), jnp.float32)),
    mesh=plsc.VectorSubcoreMesh(core_axis_name="c", subcore_axis_name="s"),
    compiler_params=pltpu.CompilerParams(use_tc_tiling_on_sc=False),
    scratch_types=[
        pltpu.VMEM((CH1 // SUB, SUB), jnp.int32),
        pltpu.VMEM((CH1 // SUB, SUB), jnp.int32),
        pltpu.VMEM((CH1, HID), jnp.float32),
        pltpu.VMEM((CH1, HID), jnp.float32),
        pltpu.VMEM((CH1, HID), jnp.float32),
        pltpu.VMEM((2, HID), jnp.float32),
        pltpu.SemaphoreType.DMA,
    ],
)(_s1_body)


def _dense(p, v):
    return v @ p['w'] + p['b']


def _bn(p, v):
    m = jnp.mean(v, axis=0)
    va = jnp.var(v, axis=0)
    return (v - m) * jax.lax.rsqrt(va + 1e-5) * p['g'] + p['b']


def _scorer_block(cat_ref, w1_ref, b1_ref, w2_ref, b2_ref, out_ref):
    sblk = jnp.maximum(cat_ref[...] @ w1_ref[...] + b1_ref[...], 0.0)
    out_ref[...] = sblk @ w2_ref[...] + b2_ref[...]


def _scorer(cat, w1, b1, w2, b2):
    BLK = 2560
    grid = (E // BLK,)
    return pl.pallas_call(
        _scorer_block,
        grid=grid,
        in_specs=[
            pl.BlockSpec((BLK, 3 * HID), lambda i: (i, 0)),
            pl.BlockSpec((3 * HID, HID), lambda i: (0, 0)),
            pl.BlockSpec((1, HID), lambda i: (0, 0)),
            pl.BlockSpec((HID, 1), lambda i: (0, 0)),
            pl.BlockSpec((1, 1), lambda i: (0, 0)),
        ],
        out_specs=pl.BlockSpec((BLK, 1), lambda i: (i, 0)),
        out_shape=jax.ShapeDtypeStruct((E, 1), jnp.float32),
    )(cat, w1, b1[None, :], w2, b2[None, :])


def _layer(p, src, dst, h, e):
    h_in, e_in = h, e
    A1h = _dense(p['A1'], h)
    A2h = _dense(p['A2'], h)
    A3h = _dense(p['A3'], h)
    B1h = _dense(p['B1'], h)
    B2h = _dense(p['B2'], h)
    B3e = _dense(p['B3'], e)
    t, stats = _s1_call(B1h, B2h, B3e, src, dst)
    mean = jnp.sum(stats[:, 0, :], axis=0) / E
    var = jnp.sum(stats[:, 1, :], axis=0) / E - mean * mean
    a = p['bn_e']['g'] * jax.lax.rsqrt(var + 1e-5)
    sh = p['bn_e']['b'] - mean * a
    bnp = jnp.stack([a, sh])
    e_new, fh, fs = _s2_call(t, e_in, A2h, src, dst, bnp)
    _, bh, bs = _s2_call(t, e_in, A3h, dst, src, bnp)
    sigma = jax.nn.sigmoid(e_new)
    fh = jax.ops.segment_sum(sigma * A2h[src], dst, num_segments=N)
    bh = jax.ops.segment_sum(sigma * A3h[dst], src, num_segments=N)
    fs = jax.ops.segment_sum(sigma, dst, num_segments=N)
    bs = jax.ops.segment_sum(sigma, src, num_segments=N)
    h_f = fh / (fs + EPS)
    h_b = bh / (bs + EPS)
    h_new = A1h + h_f + h_b
    h_new = _bn(p['bn_h'], h_new)
    h_new = jax.nn.relu(h_new)
    h_new = h_in + h_new
    return h_new, e_new


def kernel(edge_index, x, e, pe, params):
    src = edge_index[0]
    dst = edge_index[1]
    h = _dense(params['linear_pe'], pe)
    eh = jax.nn.relu(_dense(params['lin1_edge'], e))
    eh = _dense(params['lin2_edge'], eh)
    for p in params['gnn']:
        h, eh = _layer(p, src, dst, h, eh)
    cat = jnp.concatenate([h[src], h[dst], eh], axis=1)
    return _scorer(cat, params['W1']['w'], params['W1']['b'],
                   params['W2']['w'], params['W2']['b'])


# speedup vs baseline: 2.4175x; 2.4175x over previous
import functools

import jax
import jax.numpy as jnp
from jax import lax
from jax.experimental import pallas as pl
from jax.experimental.pallas import tpu as pltpu
from jax.experimental.pallas import tpu_sc as plsc

N = 10000
E = 320000
HID = 64
EPS = 1e-6

NC = 2
NS = 16
NW = NC * NS
LANES = 16

CH1 = 512
SUB = 128
NCHUNKS1 = E // CH1
TRIP1 = (NCHUNKS1 + NW - 1) // NW


def _s1_body(b1h, b2h, b3e, srcv, dstv, t_out, stats_out,
             idx_s, idx_d, g1, g2, b3, stats_v, sem):
    c = lax.axis_index("c")
    s = lax.axis_index("s")
    wid = s * NC + c

    for k in range(4):
        stats_v[0, pl.ds(16 * k, 16)] = jnp.zeros((16,), jnp.float32)
        stats_v[1, pl.ds(16 * k, 16)] = jnp.zeros((16,), jnp.float32)

    def chunk(i, carry):
        ci = wid + NW * i

        @pl.when(ci < NCHUNKS1)
        def _():
            base = ci * CH1
            for j in range(CH1 // SUB):
                pltpu.sync_copy(srcv.at[pl.ds(base + SUB * j, SUB)], idx_s.at[j])
                pltpu.sync_copy(dstv.at[pl.ds(base + SUB * j, SUB)], idx_d.at[j])
            copies = []
            for j in range(CH1 // SUB):
                copies.append(pltpu.async_copy(
                    b1h.at[idx_s.at[j]], g1.at[pl.ds(SUB * j, SUB)], sem))
                copies.append(pltpu.async_copy(
                    b2h.at[idx_d.at[j]], g2.at[pl.ds(SUB * j, SUB)], sem))
            pltpu.sync_copy(b3e.at[pl.ds(base, CH1), :], b3)
            for cp in copies:
                cp.wait()

            def edge(e, carry):
                acc = list(carry)
                for k in range(4):
                    v = g1[e, pl.ds(16 * k, 16)] + g2[e, pl.ds(16 * k, 16)] \
                        + b3[e, pl.ds(16 * k, 16)]
                    b3[e, pl.ds(16 * k, 16)] = v
                    acc[k] = acc[k] + v
                    acc[4 + k] = acc[4 + k] + v * v
                return tuple(acc)

            init = tuple(jnp.zeros((16,), jnp.float32) for _ in range(8))
            acc = lax.fori_loop(0, CH1, edge, init)
            for k in range(4):
                stats_v[0, pl.ds(16 * k, 16)] += acc[k]
                stats_v[1, pl.ds(16 * k, 16)] += acc[4 + k]
            pltpu.sync_copy(b3, t_out.at[pl.ds(base, CH1), :])
        return carry

    lax.fori_loop(0, TRIP1, chunk, 0)
    pltpu.sync_copy(stats_v, stats_out.at[wid])


CH2 = 128
NCHUNKS2 = E // CH2
TRIP2 = (NCHUNKS2 + NS - 1) // NS
NSTRIPE = N // NS
ZB = 125


def _s2_body(t_in, e_in, tbl, gidx, sidx, bnp,
             e_new, acc_h_out, acc_s_out,
             idx_g, idx_sc, g, tb, eb, sg, bn_v, zb, acc_h, acc_s, sem):
    c = lax.axis_index("c")
    s = lax.axis_index("s")

    def zrow(r, carry):
        for k in range(4):
            zb[r, pl.ds(16 * k, 16)] = jnp.zeros((16,), jnp.float32)
        return carry
    lax.fori_loop(0, ZB, zrow, 0)
    for q in range(NSTRIPE // ZB):
        rows = pl.ds(NSTRIPE * s + ZB * q, ZB)
        pltpu.sync_copy(zb, acc_h.at[rows, :])
        pltpu.sync_copy(zb, acc_s.at[rows, :])
    plsc.subcore_barrier()

    pltpu.sync_copy(bnp, bn_v)
    av = [bn_v[0, pl.ds(16 * k, 16)] for k in range(4)]
    bv = [bn_v[1, pl.ds(16 * k, 16)] for k in range(4)]

    def chunk(i, carry):
        ci = s + NS * i

        @pl.when(ci < NCHUNKS2)
        def _():
            base = ci * CH2
            for j in range(CH2 // SUB):
                pltpu.sync_copy(gidx.at[c, pl.ds(base + SUB * j, SUB)],
                                idx_g.at[j])
                pltpu.sync_copy(sidx.at[c, pl.ds(base + SUB * j, SUB)],
                                idx_sc.at[j])
            copies = [pltpu.async_copy(tbl.at[idx_g.at[j]],
                                       g.at[pl.ds(SUB * j, SUB)], sem)
                      for j in range(CH2 // SUB)]
            pltpu.sync_copy(t_in.at[pl.ds(base, CH2), :], tb)
            pltpu.sync_copy(e_in.at[pl.ds(base, CH2), :], eb)
            for cp in copies:
                cp.wait()

            def edge(ei, cr):
                for k in range(4):
                    u = jnp.maximum(
                        tb[ei, pl.ds(16 * k, 16)] * av[k] + bv[k], 0.0)
                    en = eb[ei, pl.ds(16 * k, 16)] + u
                    eb[ei, pl.ds(16 * k, 16)] = en
                    sig = 1.0 / (1.0 + jnp.exp(-en))
                    sg[ei, pl.ds(16 * k, 16)] = sig
                    g[ei, pl.ds(16 * k, 16)] = sig * g[ei, pl.ds(16 * k, 16)]
                return cr
            lax.fori_loop(0, CH2, edge, 0)

            @pl.when(c == 0)
            def _():
                pltpu.sync_copy(eb, e_new.at[pl.ds(base, CH2), :])
            for j in range(CH2 // SUB):
                pltpu.sync_copy(g.at[pl.ds(SUB * j, SUB)],
                                acc_h.at[idx_sc.at[j]], add=True)
                pltpu.sync_copy(sg.at[pl.ds(SUB * j, SUB)],
                                acc_s.at[idx_sc.at[j]], add=True)
        return carry
    lax.fori_loop(0, TRIP2, chunk, 0)

    plsc.subcore_barrier()

    rows = pl.ds(NSTRIPE * s, NSTRIPE)
    pltpu.sync_copy(acc_h.at[rows, :], acc_h_out.at[c, rows, :])
    pltpu.sync_copy(acc_s.at[rows, :], acc_s_out.at[c, rows, :])


_s2_call = functools.partial(
    pl.kernel,
    out_type=(jax.ShapeDtypeStruct((E, HID), jnp.float32),
              jax.ShapeDtypeStruct((NC, N, HID), jnp.float32),
              jax.ShapeDtypeStruct((NC, N, HID), jnp.float32)),
    mesh=plsc.VectorSubcoreMesh(core_axis_name="c", subcore_axis_name="s"),
    compiler_params=pltpu.CompilerParams(use_tc_tiling_on_sc=False),
    scratch_types=[
        pltpu.VMEM((CH2 // SUB, SUB), jnp.int32),
        pltpu.VMEM((CH2 // SUB, SUB), jnp.int32),
        pltpu.VMEM((CH2, HID), jnp.float32),
        pltpu.VMEM((CH2, HID), jnp.float32),
        pltpu.VMEM((CH2, HID), jnp.float32),
        pltpu.VMEM((CH2, HID), jnp.float32),
        pltpu.VMEM((2, HID), jnp.float32),
        pltpu.VMEM((ZB, HID), jnp.float32),
        pltpu.VMEM_SHARED((N, HID), jnp.float32),
        pltpu.VMEM_SHARED((N, HID), jnp.float32),
        pltpu.SemaphoreType.DMA,
    ],
)(_s2_body)


_s1_call = functools.partial(
    pl.kernel,
    out_type=(jax.ShapeDtypeStruct((E, HID), jnp.float32),
              jax.ShapeDtypeStruct((NW, 2, HID), jnp.float32)),
    mesh=plsc.VectorSubcoreMesh(core_axis_name="c", subcore_axis_name="s"),
    compiler_params=pltpu.CompilerParams(use_tc_tiling_on_sc=False),
    scratch_types=[
        pltpu.VMEM((CH1 // SUB, SUB), jnp.int32),
        pltpu.VMEM((CH1 // SUB, SUB), jnp.int32),
        pltpu.VMEM((CH1, HID), jnp.float32),
        pltpu.VMEM((CH1, HID), jnp.float32),
        pltpu.VMEM((CH1, HID), jnp.float32),
        pltpu.VMEM((2, HID), jnp.float32),
        pltpu.SemaphoreType.DMA,
    ],
)(_s1_body)


def _dense(p, v):
    return v @ p['w'] + p['b']


def _bn(p, v):
    m = jnp.mean(v, axis=0)
    va = jnp.var(v, axis=0)
    return (v - m) * jax.lax.rsqrt(va + 1e-5) * p['g'] + p['b']


def _scorer_block(cat_ref, w1_ref, b1_ref, w2_ref, b2_ref, out_ref):
    sblk = jnp.maximum(cat_ref[...] @ w1_ref[...] + b1_ref[...], 0.0)
    out_ref[...] = sblk @ w2_ref[...] + b2_ref[...]


def _scorer(cat, w1, b1, w2, b2):
    BLK = 2560
    grid = (E // BLK,)
    return pl.pallas_call(
        _scorer_block,
        grid=grid,
        in_specs=[
            pl.BlockSpec((BLK, 3 * HID), lambda i: (i, 0)),
            pl.BlockSpec((3 * HID, HID), lambda i: (0, 0)),
            pl.BlockSpec((1, HID), lambda i: (0, 0)),
            pl.BlockSpec((HID, 1), lambda i: (0, 0)),
            pl.BlockSpec((1, 1), lambda i: (0, 0)),
        ],
        out_specs=pl.BlockSpec((BLK, 1), lambda i: (i, 0)),
        out_shape=jax.ShapeDtypeStruct((E, 1), jnp.float32),
    )(cat, w1, b1[None, :], w2, b2[None, :])


def _layer(p, src, dst, gidx, sidx, h, e):
    h_in, e_in = h, e
    A1h = _dense(p['A1'], h)
    A2h = _dense(p['A2'], h)
    A3h = _dense(p['A3'], h)
    B1h = _dense(p['B1'], h)
    B2h = _dense(p['B2'], h)
    B3e = _dense(p['B3'], e)
    t, stats = _s1_call(B1h, B2h, B3e, src, dst)
    mean = jnp.sum(stats[:, 0, :], axis=0) / E
    var = jnp.sum(stats[:, 1, :], axis=0) / E - mean * mean
    a = p['bn_e']['g'] * jax.lax.rsqrt(var + 1e-5)
    sh = p['bn_e']['b'] - mean * a
    bnp = jnp.stack([a, sh])
    tbl = jnp.concatenate([A2h, A3h], axis=0)
    e_new, acc_h, acc_s = _s2_call(t, e_in, tbl, gidx, sidx, bnp)
    h_f = acc_h[0] / (acc_s[0] + EPS)
    h_b = acc_h[1] / (acc_s[1] + EPS)
    h_new = A1h + h_f + h_b
    h_new = _bn(p['bn_h'], h_new)
    h_new = jax.nn.relu(h_new)
    h_new = h_in + h_new
    return h_new, e_new


def kernel(edge_index, x, e, pe, params):
    src = edge_index[0]
    dst = edge_index[1]
    gidx = jnp.stack([src, dst + N])
    sidx = jnp.stack([dst, src])
    h = _dense(params['linear_pe'], pe)
    eh = jax.nn.relu(_dense(params['lin1_edge'], e))
    eh = _dense(params['lin2_edge'], eh)
    for p in params['gnn']:
        h, eh = _layer(p, src, dst, gidx, sidx, h, eh)
    cat = jnp.concatenate([h[src], h[dst], eh], axis=1)
    return _scorer(cat, params['W1']['w'], params['W1']['b'],
                   params['W2']['w'], params['W2']['b'])
